# baseline (speedup 1.0000x reference)
import functools

import jax
import jax.numpy as jnp
from jax import lax
from jax.experimental import pallas as pl
from jax.experimental.pallas import tpu as pltpu
from jax.experimental.pallas import tpu_sc as plsc

_NC = 2
_NS = 16
_F32 = jnp.float32


def _combine_body(basis_ref, comp_ref, out_ref):
    r = pl.program_id(1)
    b_dim = basis_ref.shape[0]
    acc = comp_ref[r, 0] * basis_ref[0]
    for b in range(1, b_dim):
        acc = acc + comp_ref[r, b] * basis_ref[b]
    out_ref[...] = acc


def _materialize_weight1(basis1, comp1, nb):
    b_dim, n, h = basis1.shape
    r_dim = comp1.shape[0]
    nblk = n // nb
    return pl.pallas_call(
        _combine_body,
        grid=(nblk, r_dim),
        in_specs=[
            pl.BlockSpec((b_dim, nb, h), lambda i, r: (0, i, 0)),
            pl.BlockSpec(memory_space=pltpu.SMEM),
        ],
        out_specs=pl.BlockSpec((nb, h), lambda i, r: (r * nblk + i, 0)),
        out_shape=jax.ShapeDtypeStruct((r_dim * n, h), _F32),
    )(basis1, comp1)


def _sc_agg(table, gidx, dst, n_nodes, width, with_cnt, chunk):
    e = gidx.shape[0]
    nw = _NC * _NS
    per_w = e // nw
    nchunk = per_w // chunk
    assert per_w % chunk == 0 and e % nw == 0
    rows_pt = (-(-n_nodes // _NS) + 7) // 8 * 8
    n_pad = rows_pt * _NS
    zrows = 136
    assert rows_pt % zrows == 0
    cpt = (-(-n_nodes // _NS) + 127) // 128 * 128
    n_cnt = cpt * _NS

    mesh = plsc.VectorSubcoreMesh(core_axis_name="c", subcore_axis_name="s",
                                  num_cores=_NC, num_subcores=_NS)
    out_type = [jax.ShapeDtypeStruct((_NC, n_pad, width), _F32)]
    if with_cnt:
        out_type.append(jax.ShapeDtypeStruct((_NC * n_cnt,), _F32))

    scratch = dict(
        idx_v=pltpu.VMEM((chunk,), jnp.int32),
        dst_v=pltpu.VMEM((chunk,), jnp.int32),
        rows_v=pltpu.VMEM((chunk, width), _F32),
        zbuf=pltpu.VMEM((zrows, width), _F32),
        agg_sh=pltpu.VMEM_SHARED((n_pad, width), _F32),
        sem=pltpu.SemaphoreType.DMA,
    )
    if with_cnt:
        scratch.update(
            ones_v=pltpu.VMEM((1024,), _F32),
            zflat=pltpu.VMEM((1024,), _F32),
            cnt_sh=pltpu.VMEM_SHARED((n_cnt,), _F32),
        )

    def body(table_r, gidx_r, dst_r, *outs, idx_v, dst_v, rows_v, zbuf,
             agg_sh, sem, ones_v=None, zflat=None, cnt_sh=None):
        if with_cnt:
            agg_out, cnt_out = outs
        else:
            (agg_out,) = outs
        c = lax.axis_index("c")
        s = lax.axis_index("s")
        wid = c * _NS + s

        z16 = jnp.zeros((16,), _F32)
        for i in range(zrows):
            for j in range(width // 16):
                zbuf[i, j * 16:(j + 1) * 16] = z16

        def zero_rows(k, _):
            row0 = s * rows_pt + k * zrows
            pltpu.sync_copy(zbuf, agg_sh.at[pl.ds(row0, zrows)])
            return 0
        lax.fori_loop(0, rows_pt // zrows, zero_rows, 0)

        if with_cnt:
            one16 = jnp.ones((16,), _F32)

            def fill(k, _):
                zflat[pl.ds(k * 16, 16)] = z16
                ones_v[pl.ds(k * 16, 16)] = one16
                return 0
            lax.fori_loop(0, 64, fill, 0)

            nfull, rem = cpt // 1024, cpt % 1024

            def zero_cnt(k, _):
                pltpu.sync_copy(zflat, cnt_sh.at[pl.ds(s * cpt + k * 1024, 1024)])
                return 0
            lax.fori_loop(0, nfull, zero_cnt, 0)
            if rem:
                pltpu.sync_copy(zflat.at[pl.ds(0, rem)],
                                cnt_sh.at[pl.ds(s * cpt + nfull * 1024, rem)])

        plsc.subcore_barrier()

        def chunk_body(j, _):
            base = wid * per_w + j * chunk
            pltpu.sync_copy(gidx_r.at[pl.ds(base, chunk)], idx_v)
            pltpu.sync_copy(dst_r.at[pl.ds(base, chunk)], dst_v)
            pltpu.async_copy(table_r.at[idx_v], rows_v, sem).wait()
            pltpu.sync_copy(rows_v, agg_sh.at[dst_v], add=True)
            if with_cnt:
                pltpu.sync_copy(ones_v.at[pl.ds(0, chunk)],
                                cnt_sh.at[dst_v], add=True)
            return 0
        lax.fori_loop(0, nchunk, chunk_body, 0)

        plsc.subcore_barrier()

        row0 = s * rows_pt
        pltpu.sync_copy(agg_sh.at[pl.ds(row0, rows_pt)],
                        agg_out.at[c, pl.ds(row0, rows_pt)])
        if with_cnt:
            pltpu.sync_copy(cnt_sh.at[pl.ds(s * cpt, cpt)],
                            cnt_out.at[pl.ds(c * n_cnt + s * cpt, cpt)])

    run = pl.kernel(body, out_type=out_type, mesh=mesh,
                    scratch_types=scratch,
                    compiler_params=pltpu.CompilerParams(
                        use_tc_tiling_on_sc=False))
    return run(table, gidx, dst)


def _hidden_body(agg_ref, cnt_ref, root_ref, bias_ref, h_ref):
    denom = jnp.maximum(cnt_ref[0] + cnt_ref[1], 1.0)
    h = (agg_ref[0] + agg_ref[1]) / denom + root_ref[...] + bias_ref[...]
    h_ref[...] = jnp.maximum(h, 0.0)


def _hidden(agg1p, cntp, root1, bias1, nb):
    n, h_dim = root1.shape
    return pl.pallas_call(
        _hidden_body,
        grid=(n // nb,),
        in_specs=[
            pl.BlockSpec((2, nb, h_dim), lambda i: (0, i, 0)),
            pl.BlockSpec((2, nb, 1), lambda i: (0, i, 0)),
            pl.BlockSpec((nb, h_dim), lambda i: (i, 0)),
            pl.BlockSpec((1, h_dim), lambda i: (0, 0)),
        ],
        out_specs=pl.BlockSpec((nb, h_dim), lambda i: (i, 0)),
        out_shape=jax.ShapeDtypeStruct((n, h_dim), _F32),
    )(agg1p, cntp, root1, bias1)


def _table2_body(h_ref, comp2_ref, basis2_ref, out_ref):
    r = pl.program_id(1)
    b_dim = basis2_ref.shape[0]
    w2r = comp2_ref[r, 0] * basis2_ref[0]
    for b in range(1, b_dim):
        w2r = w2r + comp2_ref[r, b] * basis2_ref[b]
    out_ref[...] = jnp.dot(h_ref[...], w2r, preferred_element_type=_F32)


def _table2(h, comp2, basis2, nb):
    n, h_dim = h.shape
    r_dim = comp2.shape[0]
    c_dim = basis2.shape[2]
    b_dim = basis2.shape[0]
    nblk = n // nb
    return pl.pallas_call(
        _table2_body,
        grid=(nblk, r_dim),
        in_specs=[
            pl.BlockSpec((nb, h_dim), lambda i, r: (i, 0)),
            pl.BlockSpec(memory_space=pltpu.SMEM),
            pl.BlockSpec((b_dim, h_dim, c_dim), lambda i, r: (0, 0, 0)),
        ],
        out_specs=pl.BlockSpec((nb, c_dim), lambda i, r: (r * nblk + i, 0)),
        out_shape=jax.ShapeDtypeStruct((r_dim * n, c_dim), _F32),
    )(h, comp2, basis2)


def _out_body(agg_ref, cnt_ref, h_ref, root2_ref, bias_ref, out_ref):
    denom = jnp.maximum(cnt_ref[0] + cnt_ref[1], 1.0)
    o = (agg_ref[0] + agg_ref[1]) / denom
    o = o + jnp.dot(h_ref[...], root2_ref[...], preferred_element_type=_F32)
    o = o + bias_ref[...]
    m = jnp.max(o, axis=1, keepdims=True)
    e = o - m
    out_ref[...] = e - jnp.log(jnp.sum(jnp.exp(e), axis=1, keepdims=True))


def _final_out(agg2p, cntp, h, root2, bias2, nb):
    n, h_dim = h.shape
    c_dim = root2.shape[1]
    return pl.pallas_call(
        _out_body,
        grid=(n // nb,),
        in_specs=[
            pl.BlockSpec((2, nb, c_dim), lambda i: (0, i, 0)),
            pl.BlockSpec((2, nb, 1), lambda i: (0, i, 0)),
            pl.BlockSpec((nb, h_dim), lambda i: (i, 0)),
            pl.BlockSpec((h_dim, c_dim), lambda i: (0, 0)),
            pl.BlockSpec((1, c_dim), lambda i: (0, 0)),
        ],
        out_specs=pl.BlockSpec((nb, c_dim), lambda i: (i, 0)),
        out_shape=jax.ShapeDtypeStruct((n, c_dim), _F32),
    )(agg2p, cntp, h, root2, bias2)


def kernel(x, edge_index, edge_type, basis1, comp1, root1, bias1,
           basis2, comp2, root2, bias2):
    del x
    b_dim, n, h_dim = basis1.shape
    r_dim = comp1.shape[0]
    c_dim = basis2.shape[2]
    e = edge_type.shape[0]

    src = edge_index[0]
    dst = edge_index[1]
    gidx = edge_type * n + src

    w1 = _materialize_weight1(basis1, comp1, nb=2000)

    agg1p, cntp = _sc_agg(w1, gidx, dst, n, h_dim, with_cnt=True, chunk=200)
    cnt3 = cntp.reshape(_NC, -1, 1)

    h = _hidden(agg1p, cnt3, root1, bias1.reshape(1, h_dim), nb=2000)
    yw = _table2(h, comp2, basis2, nb=2000)

    (agg2p,) = _sc_agg(yw, gidx, dst, n, c_dim, with_cnt=False, chunk=1000)

    return _final_out(agg2p, cnt3, h, root2, bias2.reshape(1, c_dim), nb=2000)


# speedup vs baseline: 1.9321x; 1.9321x over previous
import functools

import jax
import jax.numpy as jnp
from jax import lax
from jax.experimental import pallas as pl
from jax.experimental.pallas import tpu as pltpu
from jax.experimental.pallas import tpu_sc as plsc

_NC = 2
_NS = 16
_F32 = jnp.float32


def _combine_body(basis_ref, comp_ref, out_ref):
    r_dim = comp_ref.shape[0]
    b_dim = basis_ref.shape[0]
    for r in range(r_dim):
        acc = comp_ref[r, 0] * basis_ref[0]
        for b in range(1, b_dim):
            acc = acc + comp_ref[r, b] * basis_ref[b]
        out_ref[r] = acc


def _materialize_weight1(basis1, comp1, nbb):
    b_dim, n, h = basis1.shape
    r_dim = comp1.shape[0]
    npk = n * h // 128
    nblk = npk // nbb
    basis1p = basis1.reshape(b_dim, nblk, nbb, 128)
    out = pl.pallas_call(
        _combine_body,
        grid=(nblk,),
        in_specs=[
            pl.BlockSpec((b_dim, 1, nbb, 128), lambda i: (0, i, 0, 0)),
            pl.BlockSpec(memory_space=pltpu.SMEM),
        ],
        out_specs=pl.BlockSpec((r_dim, 1, nbb, 128), lambda i: (0, i, 0, 0)),
        out_shape=jax.ShapeDtypeStruct((r_dim, nblk, nbb, 128), _F32),
    )(basis1p, comp1)
    return out.reshape(r_dim * n, h)


def _sc_agg(table, gidx, dst, n_nodes, width, with_cnt, chunk):
    e = gidx.shape[0]
    nw = _NC * _NS
    per_w = e // nw
    nchunk = per_w // chunk
    assert per_w % chunk == 0 and e % nw == 0
    rows_pt = (-(-n_nodes // _NS) + 7) // 8 * 8
    n_pad = rows_pt * _NS
    zrows = 136
    assert rows_pt % zrows == 0
    cpt = (-(-n_nodes // _NS) + 127) // 128 * 128
    n_cnt = cpt * _NS

    mesh = plsc.VectorSubcoreMesh(core_axis_name="c", subcore_axis_name="s",
                                  num_cores=_NC, num_subcores=_NS)
    out_type = [jax.ShapeDtypeStruct((_NC, n_pad, width), _F32)]
    if with_cnt:
        out_type.append(jax.ShapeDtypeStruct((_NC * n_cnt,), _F32))

    scratch = dict(
        idx_v=pltpu.VMEM((chunk,), jnp.int32),
        dst_v=pltpu.VMEM((chunk,), jnp.int32),
        rows_v=pltpu.VMEM((chunk, width), _F32),
        zbuf=pltpu.VMEM((zrows, width), _F32),
        agg_sh=pltpu.VMEM_SHARED((n_pad, width), _F32),
        sem=pltpu.SemaphoreType.DMA,
    )
    if with_cnt:
        scratch.update(
            ones_v=pltpu.VMEM((1024,), _F32),
            zflat=pltpu.VMEM((1024,), _F32),
            cnt_sh=pltpu.VMEM_SHARED((n_cnt,), _F32),
        )

    def body(table_r, gidx_r, dst_r, *outs, idx_v, dst_v, rows_v, zbuf,
             agg_sh, sem, ones_v=None, zflat=None, cnt_sh=None):
        if with_cnt:
            agg_out, cnt_out = outs
        else:
            (agg_out,) = outs
        c = lax.axis_index("c")
        s = lax.axis_index("s")
        wid = c * _NS + s

        z16 = jnp.zeros((16,), _F32)
        for i in range(zrows):
            for j in range(width // 16):
                zbuf[i, j * 16:(j + 1) * 16] = z16

        def zero_rows(k, _):
            row0 = s * rows_pt + k * zrows
            pltpu.sync_copy(zbuf, agg_sh.at[pl.ds(row0, zrows)])
            return 0
        lax.fori_loop(0, rows_pt // zrows, zero_rows, 0)

        if with_cnt:
            one16 = jnp.ones((16,), _F32)

            def fill(k, _):
                zflat[pl.ds(k * 16, 16)] = z16
                ones_v[pl.ds(k * 16, 16)] = one16
                return 0
            lax.fori_loop(0, 64, fill, 0)

            nfull, rem = cpt // 1024, cpt % 1024

            def zero_cnt(k, _):
                pltpu.sync_copy(zflat, cnt_sh.at[pl.ds(s * cpt + k * 1024, 1024)])
                return 0
            lax.fori_loop(0, nfull, zero_cnt, 0)
            if rem:
                pltpu.sync_copy(zflat.at[pl.ds(0, rem)],
                                cnt_sh.at[pl.ds(s * cpt + nfull * 1024, rem)])

        plsc.subcore_barrier()

        def chunk_body(j, _):
            base = wid * per_w + j * chunk
            pltpu.sync_copy(gidx_r.at[pl.ds(base, chunk)], idx_v)
            pltpu.sync_copy(dst_r.at[pl.ds(base, chunk)], dst_v)
            pltpu.async_copy(table_r.at[idx_v], rows_v, sem).wait()
            pltpu.sync_copy(rows_v, agg_sh.at[dst_v], add=True)
            if with_cnt:
                pltpu.sync_copy(ones_v.at[pl.ds(0, chunk)],
                                cnt_sh.at[dst_v], add=True)
            return 0
        lax.fori_loop(0, nchunk, chunk_body, 0)

        plsc.subcore_barrier()

        row0 = s * rows_pt
        pltpu.sync_copy(agg_sh.at[pl.ds(row0, rows_pt)],
                        agg_out.at[c, pl.ds(row0, rows_pt)])
        if with_cnt:
            pltpu.sync_copy(cnt_sh.at[pl.ds(s * cpt, cpt)],
                            cnt_out.at[pl.ds(c * n_cnt + s * cpt, cpt)])

    run = pl.kernel(body, out_type=out_type, mesh=mesh,
                    scratch_types=scratch,
                    compiler_params=pltpu.CompilerParams(
                        use_tc_tiling_on_sc=False))
    return run(table, gidx, dst)


def _hidden_body(agg_ref, cnt_ref, root_ref, bias_ref, h_ref):
    denom = jnp.maximum(cnt_ref[0] + cnt_ref[1], 1.0)
    h = (agg_ref[0] + agg_ref[1]) / denom + root_ref[...] + bias_ref[...]
    h_ref[...] = jnp.maximum(h, 0.0)


def _hidden(agg1p, cntp, root1, bias1, nb):
    n, h_dim = root1.shape
    return pl.pallas_call(
        _hidden_body,
        grid=(n // nb,),
        in_specs=[
            pl.BlockSpec((2, nb, h_dim), lambda i: (0, i, 0)),
            pl.BlockSpec((2, nb, 1), lambda i: (0, i, 0)),
            pl.BlockSpec((nb, h_dim), lambda i: (i, 0)),
            pl.BlockSpec((1, h_dim), lambda i: (0, 0)),
        ],
        out_specs=pl.BlockSpec((nb, h_dim), lambda i: (i, 0)),
        out_shape=jax.ShapeDtypeStruct((n, h_dim), _F32),
    )(agg1p, cntp, root1, bias1)


def _table2_body(h_ref, comp2_ref, basis2_ref, out_ref):
    r_dim, b_dim = comp2_ref.shape
    cols = []
    for r in range(r_dim):
        w2r = comp2_ref[r, 0] * basis2_ref[0]
        for b in range(1, b_dim):
            w2r = w2r + comp2_ref[r, b] * basis2_ref[b]
        cols.append(w2r)
    w_all = jnp.concatenate(cols, axis=1)
    out_ref[...] = jnp.dot(h_ref[...], w_all, preferred_element_type=_F32)


def _table2(h, comp2, basis2, nb):
    n, h_dim = h.shape
    r_dim = comp2.shape[0]
    c_dim = basis2.shape[2]
    b_dim = basis2.shape[0]
    out = pl.pallas_call(
        _table2_body,
        grid=(n // nb,),
        in_specs=[
            pl.BlockSpec((nb, h_dim), lambda i: (i, 0)),
            pl.BlockSpec(memory_space=pltpu.SMEM),
            pl.BlockSpec((b_dim, h_dim, c_dim), lambda i: (0, 0, 0)),
        ],
        out_specs=pl.BlockSpec((nb, r_dim * c_dim), lambda i: (i, 0)),
        out_shape=jax.ShapeDtypeStruct((n, r_dim * c_dim), _F32),
    )(h, comp2, basis2)
    return out.reshape(n * r_dim, c_dim)


def _out_body(agg_ref, cnt_ref, h_ref, root2_ref, bias_ref, out_ref):
    denom = jnp.maximum(cnt_ref[0] + cnt_ref[1], 1.0)
    o = (agg_ref[0] + agg_ref[1]) / denom
    o = o + jnp.dot(h_ref[...], root2_ref[...], preferred_element_type=_F32)
    o = o + bias_ref[...]
    m = jnp.max(o, axis=1, keepdims=True)
    e = o - m
    out_ref[...] = e - jnp.log(jnp.sum(jnp.exp(e), axis=1, keepdims=True))


def _final_out(agg2p, cntp, h, root2, bias2, nb):
    n, h_dim = h.shape
    c_dim = root2.shape[1]
    return pl.pallas_call(
        _out_body,
        grid=(n // nb,),
        in_specs=[
            pl.BlockSpec((2, nb, c_dim), lambda i: (0, i, 0)),
            pl.BlockSpec((2, nb, 1), lambda i: (0, i, 0)),
            pl.BlockSpec((nb, h_dim), lambda i: (i, 0)),
            pl.BlockSpec((h_dim, c_dim), lambda i: (0, 0)),
            pl.BlockSpec((1, c_dim), lambda i: (0, 0)),
        ],
        out_specs=pl.BlockSpec((nb, c_dim), lambda i: (i, 0)),
        out_shape=jax.ShapeDtypeStruct((n, c_dim), _F32),
    )(agg2p, cntp, h, root2, bias2)


def kernel(x, edge_index, edge_type, basis1, comp1, root1, bias1,
           basis2, comp2, root2, bias2):
    del x
    b_dim, n, h_dim = basis1.shape
    r_dim = comp1.shape[0]
    c_dim = basis2.shape[2]
    e = edge_type.shape[0]

    src = edge_index[0]
    dst = edge_index[1]
    gidx1 = edge_type * n + src
    gidx2 = src * r_dim + edge_type

    w1 = _materialize_weight1(basis1, comp1, nbb=500)

    agg1p, cntp = _sc_agg(w1, gidx1, dst, n, h_dim, with_cnt=True, chunk=200)
    cnt3 = cntp.reshape(_NC, -1, 1)

    h = _hidden(agg1p, cnt3, root1, bias1.reshape(1, h_dim), nb=2000)
    yw = _table2(h, comp2, basis2, nb=2000)

    (agg2p,) = _sc_agg(yw, gidx2, dst, n, c_dim, with_cnt=False, chunk=1000)

    return _final_out(agg2p, cnt3, h, root2, bias2.reshape(1, c_dim), nb=2000)


# speedup vs baseline: 2.2817x; 1.1809x over previous
import functools

import jax
import jax.numpy as jnp
from jax import lax
from jax.experimental import pallas as pl
from jax.experimental.pallas import tpu as pltpu
from jax.experimental.pallas import tpu_sc as plsc

_NC = 2
_NS = 16
_F32 = jnp.float32


def _combine_body(basis_ref, comp_ref, out_ref):
    r_dim = comp_ref.shape[0]
    b_dim = basis_ref.shape[0]
    for r in range(r_dim):
        acc = comp_ref[r, 0] * basis_ref[0]
        for b in range(1, b_dim):
            acc = acc + comp_ref[r, b] * basis_ref[b]
        out_ref[r] = acc


def _materialize_weight1(basis1, comp1, nbb):
    b_dim, n, h = basis1.shape
    r_dim = comp1.shape[0]
    npk = n * h // 128
    nblk = npk // nbb
    basis1p = basis1.reshape(b_dim, nblk, nbb, 128)
    out = pl.pallas_call(
        _combine_body,
        grid=(nblk,),
        in_specs=[
            pl.BlockSpec((b_dim, 1, nbb, 128), lambda i: (0, i, 0, 0)),
            pl.BlockSpec(memory_space=pltpu.SMEM),
        ],
        out_specs=pl.BlockSpec((r_dim, 1, nbb, 128), lambda i: (0, i, 0, 0)),
        out_shape=jax.ShapeDtypeStruct((r_dim, nblk, nbb, 128), _F32),
    )(basis1p, comp1)
    return out.reshape(r_dim * n, h)


def _sc_agg(table, gidx, dst, n_nodes, width, with_cnt, chunk):
    e = gidx.shape[0]
    nw = _NC * _NS
    per_w = e // nw
    nchunk = per_w // chunk
    assert per_w % chunk == 0 and e % nw == 0
    rows_pt = (-(-n_nodes // _NS) + 7) // 8 * 8
    n_pad = rows_pt * _NS
    zrows = 136
    assert rows_pt % zrows == 0
    cpt = (-(-n_nodes // _NS) + 127) // 128 * 128
    n_cnt = cpt * _NS

    mesh = plsc.VectorSubcoreMesh(core_axis_name="c", subcore_axis_name="s",
                                  num_cores=_NC, num_subcores=_NS)
    out_type = [jax.ShapeDtypeStruct((_NC, n_pad, width), _F32)]
    if with_cnt:
        out_type.append(jax.ShapeDtypeStruct((_NC * n_cnt,), _F32))

    scratch = dict(
        idx_v0=pltpu.VMEM((chunk,), jnp.int32),
        idx_v1=pltpu.VMEM((chunk,), jnp.int32),
        dst_v0=pltpu.VMEM((chunk,), jnp.int32),
        dst_v1=pltpu.VMEM((chunk,), jnp.int32),
        rows_v0=pltpu.VMEM((chunk, width), _F32),
        rows_v1=pltpu.VMEM((chunk, width), _F32),
        zbuf=pltpu.VMEM((zrows, width), _F32),
        agg_sh=pltpu.VMEM_SHARED((n_pad, width), _F32),
        sem0=pltpu.SemaphoreType.DMA,
        sem1=pltpu.SemaphoreType.DMA,
    )
    if with_cnt:
        scratch.update(
            ones_v=pltpu.VMEM((1024,), _F32),
            zflat=pltpu.VMEM((1024,), _F32),
            cnt_sh=pltpu.VMEM_SHARED((n_cnt,), _F32),
        )

    def body(table_r, gidx_r, dst_r, *outs, idx_v0, idx_v1, dst_v0, dst_v1,
             rows_v0, rows_v1, zbuf, agg_sh, sem0, sem1,
             ones_v=None, zflat=None, cnt_sh=None):
        if with_cnt:
            agg_out, cnt_out = outs
        else:
            (agg_out,) = outs
        c = lax.axis_index("c")
        s = lax.axis_index("s")
        wid = c * _NS + s

        z16 = jnp.zeros((16,), _F32)
        for i in range(zrows):
            for j in range(width // 16):
                zbuf[i, j * 16:(j + 1) * 16] = z16

        def zero_rows(k, _):
            row0 = s * rows_pt + k * zrows
            pltpu.sync_copy(zbuf, agg_sh.at[pl.ds(row0, zrows)])
            return 0
        lax.fori_loop(0, rows_pt // zrows, zero_rows, 0)

        if with_cnt:
            one16 = jnp.ones((16,), _F32)

            def fill(k, _):
                zflat[pl.ds(k * 16, 16)] = z16
                ones_v[pl.ds(k * 16, 16)] = one16
                return 0
            lax.fori_loop(0, 64, fill, 0)

            nfull, rem = cpt // 1024, cpt % 1024

            def zero_cnt(k, _):
                pltpu.sync_copy(zflat, cnt_sh.at[pl.ds(s * cpt + k * 1024, 1024)])
                return 0
            lax.fori_loop(0, nfull, zero_cnt, 0)
            if rem:
                pltpu.sync_copy(zflat.at[pl.ds(0, rem)],
                                cnt_sh.at[pl.ds(s * cpt + nfull * 1024, rem)])

        plsc.subcore_barrier()

        def start(j, idx_v, dst_v, rows_v, sem):
            base = wid * per_w + j * chunk
            pltpu.sync_copy(gidx_r.at[pl.ds(base, chunk)], idx_v)
            pltpu.sync_copy(dst_r.at[pl.ds(base, chunk)], dst_v)
            pltpu.async_copy(table_r.at[idx_v], rows_v, sem)

        def drain(idx_v, rows_v, sem):
            pltpu.make_async_copy(table_r.at[idx_v], rows_v, sem).wait()

        def scatter(dst_v, rows_v):
            pltpu.sync_copy(rows_v, agg_sh.at[dst_v], add=True)
            if with_cnt:
                pltpu.sync_copy(ones_v.at[pl.ds(0, chunk)],
                                cnt_sh.at[dst_v], add=True)

        assert nchunk % 2 == 1
        start(0, idx_v0, dst_v0, rows_v0, sem0)

        def pair_body(p, _):
            start(2 * p + 1, idx_v1, dst_v1, rows_v1, sem1)
            drain(idx_v0, rows_v0, sem0)
            scatter(dst_v0, rows_v0)
            start(2 * p + 2, idx_v0, dst_v0, rows_v0, sem0)
            drain(idx_v1, rows_v1, sem1)
            scatter(dst_v1, rows_v1)
            return 0
        lax.fori_loop(0, (nchunk - 1) // 2, pair_body, 0)
        drain(idx_v0, rows_v0, sem0)
        scatter(dst_v0, rows_v0)

        plsc.subcore_barrier()

        row0 = s * rows_pt
        pltpu.sync_copy(agg_sh.at[pl.ds(row0, rows_pt)],
                        agg_out.at[c, pl.ds(row0, rows_pt)])
        if with_cnt:
            pltpu.sync_copy(cnt_sh.at[pl.ds(s * cpt, cpt)],
                            cnt_out.at[pl.ds(c * n_cnt + s * cpt, cpt)])

    run = pl.kernel(body, out_type=out_type, mesh=mesh,
                    scratch_types=scratch,
                    compiler_params=pltpu.CompilerParams(
                        use_tc_tiling_on_sc=False))
    return run(table, gidx, dst)


def _hidden_body(agg_ref, cnt_ref, root_ref, bias_ref, h_ref):
    denom = jnp.maximum(cnt_ref[0] + cnt_ref[1], 1.0)
    h = (agg_ref[0] + agg_ref[1]) / denom + root_ref[...] + bias_ref[...]
    h_ref[...] = jnp.maximum(h, 0.0)


def _hidden(agg1p, cntp, root1, bias1, nb):
    n, h_dim = root1.shape
    return pl.pallas_call(
        _hidden_body,
        grid=(n // nb,),
        in_specs=[
            pl.BlockSpec((2, nb, h_dim), lambda i: (0, i, 0)),
            pl.BlockSpec((2, nb, 1), lambda i: (0, i, 0)),
            pl.BlockSpec((nb, h_dim), lambda i: (i, 0)),
            pl.BlockSpec((1, h_dim), lambda i: (0, 0)),
        ],
        out_specs=pl.BlockSpec((nb, h_dim), lambda i: (i, 0)),
        out_shape=jax.ShapeDtypeStruct((n, h_dim), _F32),
    )(agg1p, cntp, root1, bias1)


def _table2_body(h_ref, comp2_ref, basis2_ref, out_ref):
    r_dim, b_dim = comp2_ref.shape
    cols = []
    for r in range(r_dim):
        w2r = comp2_ref[r, 0] * basis2_ref[0]
        for b in range(1, b_dim):
            w2r = w2r + comp2_ref[r, b] * basis2_ref[b]
        cols.append(w2r)
    w_all = jnp.concatenate(cols, axis=1)
    out_ref[...] = jnp.dot(h_ref[...], w_all, preferred_element_type=_F32)


def _table2(h, comp2, basis2, nb):
    n, h_dim = h.shape
    r_dim = comp2.shape[0]
    c_dim = basis2.shape[2]
    b_dim = basis2.shape[0]
    out = pl.pallas_call(
        _table2_body,
        grid=(n // nb,),
        in_specs=[
            pl.BlockSpec((nb, h_dim), lambda i: (i, 0)),
            pl.BlockSpec(memory_space=pltpu.SMEM),
            pl.BlockSpec((b_dim, h_dim, c_dim), lambda i: (0, 0, 0)),
        ],
        out_specs=pl.BlockSpec((nb, r_dim * c_dim), lambda i: (i, 0)),
        out_shape=jax.ShapeDtypeStruct((n, r_dim * c_dim), _F32),
    )(h, comp2, basis2)
    return out.reshape(n * r_dim, c_dim)


def _out_body(agg_ref, cnt_ref, h_ref, root2_ref, bias_ref, out_ref):
    denom = jnp.maximum(cnt_ref[0] + cnt_ref[1], 1.0)
    o = (agg_ref[0] + agg_ref[1]) / denom
    o = o + jnp.dot(h_ref[...], root2_ref[...], preferred_element_type=_F32)
    o = o + bias_ref[...]
    m = jnp.max(o, axis=1, keepdims=True)
    e = o - m
    out_ref[...] = e - jnp.log(jnp.sum(jnp.exp(e), axis=1, keepdims=True))


def _final_out(agg2p, cntp, h, root2, bias2, nb):
    n, h_dim = h.shape
    c_dim = root2.shape[1]
    return pl.pallas_call(
        _out_body,
        grid=(n // nb,),
        in_specs=[
            pl.BlockSpec((2, nb, c_dim), lambda i: (0, i, 0)),
            pl.BlockSpec((2, nb, 1), lambda i: (0, i, 0)),
            pl.BlockSpec((nb, h_dim), lambda i: (i, 0)),
            pl.BlockSpec((h_dim, c_dim), lambda i: (0, 0)),
            pl.BlockSpec((1, c_dim), lambda i: (0, 0)),
        ],
        out_specs=pl.BlockSpec((nb, c_dim), lambda i: (i, 0)),
        out_shape=jax.ShapeDtypeStruct((n, c_dim), _F32),
    )(agg2p, cntp, h, root2, bias2)


def kernel(x, edge_index, edge_type, basis1, comp1, root1, bias1,
           basis2, comp2, root2, bias2):
    del x
    b_dim, n, h_dim = basis1.shape
    r_dim = comp1.shape[0]
    c_dim = basis2.shape[2]
    e = edge_type.shape[0]

    src = edge_index[0]
    dst = edge_index[1]
    gidx1 = edge_type * n + src
    gidx2 = src * r_dim + edge_type

    w1 = _materialize_weight1(basis1, comp1, nbb=500)

    agg1p, cntp = _sc_agg(w1, gidx1, dst, n, h_dim, with_cnt=True, chunk=200)
    cnt3 = cntp.reshape(_NC, -1, 1)

    h = _hidden(agg1p, cnt3, root1, bias1.reshape(1, h_dim), nb=2000)
    yw = _table2(h, comp2, basis2, nb=2000)

    (agg2p,) = _sc_agg(yw, gidx2, dst, n, c_dim, with_cnt=False, chunk=1000)

    return _final_out(agg2p, cnt3, h, root2, bias2.reshape(1, c_dim), nb=2000)


# speedup vs baseline: 2.4522x; 1.0748x over previous
import functools

import jax
import jax.numpy as jnp
from jax import lax
from jax.experimental import pallas as pl
from jax.experimental.pallas import tpu as pltpu
from jax.experimental.pallas import tpu_sc as plsc

_NC = 2
_NS = 16
_F32 = jnp.float32


def _combine_body(basis_ref, comp_ref, out_ref):
    r_dim = comp_ref.shape[0]
    b_dim = basis_ref.shape[0]
    for r in range(r_dim):
        acc = comp_ref[r, 0] * basis_ref[0]
        for b in range(1, b_dim):
            acc = acc + comp_ref[r, b] * basis_ref[b]
        out_ref[r] = acc


def _materialize_weight1(basis1, comp1, nbb):
    b_dim, n, h = basis1.shape
    r_dim = comp1.shape[0]
    npk = n * h // 128
    nblk = npk // nbb
    basis1p = basis1.reshape(b_dim, nblk, nbb, 128)
    out = pl.pallas_call(
        _combine_body,
        grid=(nblk,),
        in_specs=[
            pl.BlockSpec((b_dim, 1, nbb, 128), lambda i: (0, i, 0, 0)),
            pl.BlockSpec(memory_space=pltpu.SMEM),
        ],
        out_specs=pl.BlockSpec((r_dim, 1, nbb, 128), lambda i: (0, i, 0, 0)),
        out_shape=jax.ShapeDtypeStruct((r_dim, nblk, nbb, 128), _F32),
    )(basis1p, comp1)
    return out.reshape(r_dim * n, h)


def _sc_agg(table, gidx, dst, n_nodes, width, chunk):
    e = gidx.shape[0]
    nw = _NC * _NS
    per_w = e // nw
    nchunk = per_w // chunk
    assert per_w % chunk == 0 and e % nw == 0
    rows_pt = (-(-n_nodes // _NS) + 7) // 8 * 8
    n_pad = rows_pt * _NS
    zrows = 136
    assert rows_pt % zrows == 0

    mesh = plsc.VectorSubcoreMesh(core_axis_name="c", subcore_axis_name="s",
                                  num_cores=_NC, num_subcores=_NS)
    out_type = [jax.ShapeDtypeStruct((_NC, n_pad, width), _F32)]

    scratch = dict(
        idx_v0=pltpu.VMEM((chunk,), jnp.int32),
        idx_v1=pltpu.VMEM((chunk,), jnp.int32),
        dst_v0=pltpu.VMEM((chunk,), jnp.int32),
        dst_v1=pltpu.VMEM((chunk,), jnp.int32),
        rows_v0=pltpu.VMEM((chunk, width), _F32),
        rows_v1=pltpu.VMEM((chunk, width), _F32),
        zbuf=pltpu.VMEM((zrows, width), _F32),
        agg_sh=pltpu.VMEM_SHARED((n_pad, width), _F32),
        sem0=pltpu.SemaphoreType.DMA,
        sem1=pltpu.SemaphoreType.DMA,
    )

    def body(table_r, gidx_r, dst_r, agg_out, *, idx_v0, idx_v1, dst_v0,
             dst_v1, rows_v0, rows_v1, zbuf, agg_sh, sem0, sem1):
        c = lax.axis_index("c")
        s = lax.axis_index("s")
        wid = c * _NS + s

        z16 = jnp.zeros((16,), _F32)
        for i in range(zrows):
            for j in range(width // 16):
                zbuf[i, j * 16:(j + 1) * 16] = z16

        def zero_rows(k, _):
            row0 = s * rows_pt + k * zrows
            pltpu.sync_copy(zbuf, agg_sh.at[pl.ds(row0, zrows)])
            return 0
        lax.fori_loop(0, rows_pt // zrows, zero_rows, 0)

        plsc.subcore_barrier()

        def start(j, idx_v, dst_v, rows_v, sem):
            base = wid * per_w + j * chunk
            pltpu.sync_copy(gidx_r.at[pl.ds(base, chunk)], idx_v)
            pltpu.sync_copy(dst_r.at[pl.ds(base, chunk)], dst_v)
            pltpu.async_copy(table_r.at[idx_v], rows_v, sem)

        def drain(idx_v, rows_v, sem):
            pltpu.make_async_copy(table_r.at[idx_v], rows_v, sem).wait()

        def scatter(dst_v, rows_v):
            pltpu.sync_copy(rows_v, agg_sh.at[dst_v], add=True)

        assert nchunk % 2 == 1
        start(0, idx_v0, dst_v0, rows_v0, sem0)

        def pair_body(p, _):
            start(2 * p + 1, idx_v1, dst_v1, rows_v1, sem1)
            drain(idx_v0, rows_v0, sem0)
            scatter(dst_v0, rows_v0)
            start(2 * p + 2, idx_v0, dst_v0, rows_v0, sem0)
            drain(idx_v1, rows_v1, sem1)
            scatter(dst_v1, rows_v1)
            return 0
        lax.fori_loop(0, (nchunk - 1) // 2, pair_body, 0)
        drain(idx_v0, rows_v0, sem0)
        scatter(dst_v0, rows_v0)

        plsc.subcore_barrier()

        row0 = s * rows_pt
        pltpu.sync_copy(agg_sh.at[pl.ds(row0, rows_pt)],
                        agg_out.at[c, pl.ds(row0, rows_pt)])

    run = pl.kernel(body, out_type=out_type, mesh=mesh,
                    scratch_types=scratch,
                    compiler_params=pltpu.CompilerParams(
                        use_tc_tiling_on_sc=False))
    return run(table, gidx, dst)


def _sc_cnt(dst, n_nodes, chunk):
    e = dst.shape[0]
    nw = _NC * _NS
    per_w = e // nw
    nchunk = per_w // chunk
    assert per_w % chunk == 0 and e % nw == 0 and chunk <= 1024
    cpt = (-(-n_nodes // _NS) + 127) // 128 * 128
    n_cnt = cpt * _NS

    mesh = plsc.VectorSubcoreMesh(core_axis_name="c", subcore_axis_name="s",
                                  num_cores=_NC, num_subcores=_NS)

    def body(dst_r, cnt_out, *, dst_v0, dst_v1, ones_v, zflat, cnt_sh):
        c = lax.axis_index("c")
        s = lax.axis_index("s")
        wid = c * _NS + s

        z16 = jnp.zeros((16,), _F32)
        one16 = jnp.ones((16,), _F32)

        def fill(k, _):
            zflat[pl.ds(k * 16, 16)] = z16
            ones_v[pl.ds(k * 16, 16)] = one16
            return 0
        lax.fori_loop(0, 64, fill, 0)

        nfull, rem = cpt // 1024, cpt % 1024

        def zero_cnt(k, _):
            pltpu.sync_copy(zflat, cnt_sh.at[pl.ds(s * cpt + k * 1024, 1024)])
            return 0
        lax.fori_loop(0, nfull, zero_cnt, 0)
        if rem:
            pltpu.sync_copy(zflat.at[pl.ds(0, rem)],
                            cnt_sh.at[pl.ds(s * cpt + nfull * 1024, rem)])

        plsc.subcore_barrier()

        assert nchunk % 2 == 1
        pltpu.sync_copy(dst_r.at[pl.ds(wid * per_w, chunk)], dst_v0)

        def pair_body(p, _):
            base = wid * per_w + (2 * p + 1) * chunk
            pltpu.sync_copy(dst_r.at[pl.ds(base, chunk)], dst_v1)
            pltpu.sync_copy(ones_v.at[pl.ds(0, chunk)],
                            cnt_sh.at[dst_v0], add=True)
            pltpu.sync_copy(dst_r.at[pl.ds(base + chunk, chunk)], dst_v0)
            pltpu.sync_copy(ones_v.at[pl.ds(0, chunk)],
                            cnt_sh.at[dst_v1], add=True)
            return 0
        lax.fori_loop(0, (nchunk - 1) // 2, pair_body, 0)
        pltpu.sync_copy(ones_v.at[pl.ds(0, chunk)],
                        cnt_sh.at[dst_v0], add=True)

        plsc.subcore_barrier()
        pltpu.sync_copy(cnt_sh.at[pl.ds(s * cpt, cpt)],
                        cnt_out.at[pl.ds(c * n_cnt + s * cpt, cpt)])

    run = pl.kernel(
        body,
        out_type=[jax.ShapeDtypeStruct((_NC * n_cnt,), _F32)],
        mesh=mesh,
        scratch_types=dict(
            dst_v0=pltpu.VMEM((chunk,), jnp.int32),
            dst_v1=pltpu.VMEM((chunk,), jnp.int32),
            ones_v=pltpu.VMEM((1024,), _F32),
            zflat=pltpu.VMEM((1024,), _F32),
            cnt_sh=pltpu.VMEM_SHARED((n_cnt,), _F32),
        ),
        compiler_params=pltpu.CompilerParams(use_tc_tiling_on_sc=False))
    (cnt_out,) = run(dst)
    return cnt_out


def _hidden_body(agg_ref, cnt_ref, root_ref, bias_ref, h_ref):
    denom = jnp.maximum(cnt_ref[0] + cnt_ref[1], 1.0)
    h = (agg_ref[0] + agg_ref[1]) / denom + root_ref[...] + bias_ref[...]
    h_ref[...] = jnp.maximum(h, 0.0)


def _hidden(agg1p, cntp, root1, bias1, nb):
    n, h_dim = root1.shape
    return pl.pallas_call(
        _hidden_body,
        grid=(n // nb,),
        in_specs=[
            pl.BlockSpec((2, nb, h_dim), lambda i: (0, i, 0)),
            pl.BlockSpec((2, nb, 1), lambda i: (0, i, 0)),
            pl.BlockSpec((nb, h_dim), lambda i: (i, 0)),
            pl.BlockSpec((1, h_dim), lambda i: (0, 0)),
        ],
        out_specs=pl.BlockSpec((nb, h_dim), lambda i: (i, 0)),
        out_shape=jax.ShapeDtypeStruct((n, h_dim), _F32),
    )(agg1p, cntp, root1, bias1)


def _table2_body(h_ref, comp2_ref, basis2_ref, out_ref):
    r_dim, b_dim = comp2_ref.shape
    cols = []
    for r in range(r_dim):
        w2r = comp2_ref[r, 0] * basis2_ref[0]
        for b in range(1, b_dim):
            w2r = w2r + comp2_ref[r, b] * basis2_ref[b]
        cols.append(w2r)
    w_all = jnp.concatenate(cols, axis=1)
    out_ref[...] = jnp.dot(h_ref[...], w_all, preferred_element_type=_F32)


def _table2(h, comp2, basis2, nb):
    n, h_dim = h.shape
    r_dim = comp2.shape[0]
    c_dim = basis2.shape[2]
    b_dim = basis2.shape[0]
    out = pl.pallas_call(
        _table2_body,
        grid=(n // nb,),
        in_specs=[
            pl.BlockSpec((nb, h_dim), lambda i: (i, 0)),
            pl.BlockSpec(memory_space=pltpu.SMEM),
            pl.BlockSpec((b_dim, h_dim, c_dim), lambda i: (0, 0, 0)),
        ],
        out_specs=pl.BlockSpec((nb, r_dim * c_dim), lambda i: (i, 0)),
        out_shape=jax.ShapeDtypeStruct((n, r_dim * c_dim), _F32),
    )(h, comp2, basis2)
    return out.reshape(n * r_dim, c_dim)


def _out_body(agg_ref, cnt_ref, h_ref, root2_ref, bias_ref, out_ref):
    denom = jnp.maximum(cnt_ref[0] + cnt_ref[1], 1.0)
    o = (agg_ref[0] + agg_ref[1]) / denom
    o = o + jnp.dot(h_ref[...], root2_ref[...], preferred_element_type=_F32)
    o = o + bias_ref[...]
    m = jnp.max(o, axis=1, keepdims=True)
    e = o - m
    out_ref[...] = e - jnp.log(jnp.sum(jnp.exp(e), axis=1, keepdims=True))


def _final_out(agg2p, cntp, h, root2, bias2, nb):
    n, h_dim = h.shape
    c_dim = root2.shape[1]
    return pl.pallas_call(
        _out_body,
        grid=(n // nb,),
        in_specs=[
            pl.BlockSpec((2, nb, c_dim), lambda i: (0, i, 0)),
            pl.BlockSpec((2, nb, 1), lambda i: (0, i, 0)),
            pl.BlockSpec((nb, h_dim), lambda i: (i, 0)),
            pl.BlockSpec((h_dim, c_dim), lambda i: (0, 0)),
            pl.BlockSpec((1, c_dim), lambda i: (0, 0)),
        ],
        out_specs=pl.BlockSpec((nb, c_dim), lambda i: (i, 0)),
        out_shape=jax.ShapeDtypeStruct((n, c_dim), _F32),
    )(agg2p, cntp, h, root2, bias2)


def kernel(x, edge_index, edge_type, basis1, comp1, root1, bias1,
           basis2, comp2, root2, bias2):
    del x
    b_dim, n, h_dim = basis1.shape
    r_dim = comp1.shape[0]
    c_dim = basis2.shape[2]
    e = edge_type.shape[0]

    src = edge_index[0]
    dst = edge_index[1]
    gidx1 = edge_type * n + src
    gidx2 = src * r_dim + edge_type

    cntp = _sc_cnt(dst, n, chunk=1000)
    cnt3 = cntp.reshape(_NC, -1, 1)

    w1 = _materialize_weight1(basis1, comp1, nbb=500)

    (agg1p,) = _sc_agg(w1, gidx1, dst, n, h_dim, chunk=200)

    h = _hidden(agg1p, cnt3, root1, bias1.reshape(1, h_dim), nb=2000)
    yw = _table2(h, comp2, basis2, nb=2000)

    (agg2p,) = _sc_agg(yw, gidx2, dst, n, c_dim, chunk=1000)

    return _final_out(agg2p, cnt3, h, root2, bias2.reshape(1, c_dim), nb=2000)


# speedup vs baseline: 2.4550x; 1.0011x over previous
import functools

import jax
import jax.numpy as jnp
from jax import lax
from jax.experimental import pallas as pl
from jax.experimental.pallas import tpu as pltpu
from jax.experimental.pallas import tpu_sc as plsc

_NC = 2
_NS = 16
_F32 = jnp.float32


def _combine_body(basis_ref, comp_ref, out_ref):
    r_dim = comp_ref.shape[0]
    b_dim = basis_ref.shape[0]
    for r in range(r_dim):
        acc = comp_ref[r, 0] * basis_ref[0]
        for b in range(1, b_dim):
            acc = acc + comp_ref[r, b] * basis_ref[b]
        out_ref[r] = acc


def _materialize_weight1(basis1, comp1, nbb):
    b_dim, n, h = basis1.shape
    r_dim = comp1.shape[0]
    npk = n * h // 128
    nblk = npk // nbb
    basis1p = basis1.reshape(b_dim, nblk, nbb, 128)
    out = pl.pallas_call(
        _combine_body,
        grid=(nblk,),
        in_specs=[
            pl.BlockSpec((b_dim, 1, nbb, 128), lambda i: (0, i, 0, 0)),
            pl.BlockSpec(memory_space=pltpu.SMEM),
        ],
        out_specs=pl.BlockSpec((r_dim, 1, nbb, 128), lambda i: (0, i, 0, 0)),
        out_shape=jax.ShapeDtypeStruct((r_dim, nblk, nbb, 128), _F32),
    )(basis1p, comp1)
    return out.reshape(r_dim * n, h)


def _sc_agg(table, gidx, dst, n_nodes, width, chunk, nbuf=2):
    e = gidx.shape[0]
    nw = _NC * _NS
    per_w = e // nw
    nchunk = per_w // chunk
    assert per_w % chunk == 0 and e % nw == 0
    assert nchunk % nbuf == nbuf - 1
    rows_pt = (-(-n_nodes // _NS) + 7) // 8 * 8
    n_pad = rows_pt * _NS
    zrows = 136
    assert rows_pt % zrows == 0

    mesh = plsc.VectorSubcoreMesh(core_axis_name="c", subcore_axis_name="s",
                                  num_cores=_NC, num_subcores=_NS)
    out_type = [jax.ShapeDtypeStruct((_NC, n_pad, width), _F32)]

    scratch = dict(
        zbuf=pltpu.VMEM((zrows, width), _F32),
        agg_sh=pltpu.VMEM_SHARED((n_pad, width), _F32),
    )
    for b in range(nbuf):
        scratch[f"idx_v{b}"] = pltpu.VMEM((chunk,), jnp.int32)
        scratch[f"dst_v{b}"] = pltpu.VMEM((chunk,), jnp.int32)
        scratch[f"rows_v{b}"] = pltpu.VMEM((chunk, width), _F32)
        scratch[f"sem{b}"] = pltpu.SemaphoreType.DMA

    def body(table_r, gidx_r, dst_r, agg_out, *, zbuf, agg_sh, **bufs):
        idx_v = [bufs[f"idx_v{b}"] for b in range(nbuf)]
        dst_v = [bufs[f"dst_v{b}"] for b in range(nbuf)]
        rows_v = [bufs[f"rows_v{b}"] for b in range(nbuf)]
        sem = [bufs[f"sem{b}"] for b in range(nbuf)]
        c = lax.axis_index("c")
        s = lax.axis_index("s")
        wid = c * _NS + s

        z16 = jnp.zeros((16,), _F32)
        for i in range(zrows):
            for j in range(width // 16):
                zbuf[i, j * 16:(j + 1) * 16] = z16

        def zero_rows(k, _):
            row0 = s * rows_pt + k * zrows
            pltpu.sync_copy(zbuf, agg_sh.at[pl.ds(row0, zrows)])
            return 0
        lax.fori_loop(0, rows_pt // zrows, zero_rows, 0)

        plsc.subcore_barrier()

        def start(j, b):
            base = wid * per_w + j * chunk
            pltpu.sync_copy(gidx_r.at[pl.ds(base, chunk)], idx_v[b])
            pltpu.sync_copy(dst_r.at[pl.ds(base, chunk)], dst_v[b])
            pltpu.async_copy(table_r.at[idx_v[b]], rows_v[b], sem[b])

        def finish(b):
            pltpu.make_async_copy(table_r.at[idx_v[b]], rows_v[b],
                                  sem[b]).wait()
            pltpu.sync_copy(rows_v[b], agg_sh.at[dst_v[b]], add=True)

        for b in range(nbuf - 1):
            start(b, b)

        def ring_body(p, _):
            for k in range(nbuf):
                start(nbuf * p + nbuf - 1 + k, (nbuf - 1 + k) % nbuf)
                finish(k)
            return 0
        lax.fori_loop(0, (nchunk - nbuf + 1) // nbuf, ring_body, 0)
        for k in range(nbuf - 1):
            finish(k)

        plsc.subcore_barrier()

        row0 = s * rows_pt
        pltpu.sync_copy(agg_sh.at[pl.ds(row0, rows_pt)],
                        agg_out.at[c, pl.ds(row0, rows_pt)])

    run = pl.kernel(body, out_type=out_type, mesh=mesh,
                    scratch_types=scratch,
                    compiler_params=pltpu.CompilerParams(
                        use_tc_tiling_on_sc=False))
    return run(table, gidx, dst)


def _sc_cnt(dst, n_nodes, chunk):
    e = dst.shape[0]
    nw = _NC * _NS
    per_w = e // nw
    nchunk = per_w // chunk
    assert per_w % chunk == 0 and e % nw == 0 and chunk <= 1024
    cpt = (-(-n_nodes // _NS) + 127) // 128 * 128
    n_cnt = cpt * _NS

    mesh = plsc.VectorSubcoreMesh(core_axis_name="c", subcore_axis_name="s",
                                  num_cores=_NC, num_subcores=_NS)

    def body(dst_r, cnt_out, *, dst_v0, dst_v1, ones_v, zflat, cnt_sh):
        c = lax.axis_index("c")
        s = lax.axis_index("s")
        wid = c * _NS + s

        z16 = jnp.zeros((16,), _F32)
        one16 = jnp.ones((16,), _F32)

        def fill(k, _):
            zflat[pl.ds(k * 16, 16)] = z16
            ones_v[pl.ds(k * 16, 16)] = one16
            return 0
        lax.fori_loop(0, 64, fill, 0)

        nfull, rem = cpt // 1024, cpt % 1024

        def zero_cnt(k, _):
            pltpu.sync_copy(zflat, cnt_sh.at[pl.ds(s * cpt + k * 1024, 1024)])
            return 0
        lax.fori_loop(0, nfull, zero_cnt, 0)
        if rem:
            pltpu.sync_copy(zflat.at[pl.ds(0, rem)],
                            cnt_sh.at[pl.ds(s * cpt + nfull * 1024, rem)])

        plsc.subcore_barrier()

        assert nchunk % 2 == 1
        pltpu.sync_copy(dst_r.at[pl.ds(wid * per_w, chunk)], dst_v0)

        def pair_body(p, _):
            base = wid * per_w + (2 * p + 1) * chunk
            pltpu.sync_copy(dst_r.at[pl.ds(base, chunk)], dst_v1)
            pltpu.sync_copy(ones_v.at[pl.ds(0, chunk)],
                            cnt_sh.at[dst_v0], add=True)
            pltpu.sync_copy(dst_r.at[pl.ds(base + chunk, chunk)], dst_v0)
            pltpu.sync_copy(ones_v.at[pl.ds(0, chunk)],
                            cnt_sh.at[dst_v1], add=True)
            return 0
        lax.fori_loop(0, (nchunk - 1) // 2, pair_body, 0)
        pltpu.sync_copy(ones_v.at[pl.ds(0, chunk)],
                        cnt_sh.at[dst_v0], add=True)

        plsc.subcore_barrier()
        pltpu.sync_copy(cnt_sh.at[pl.ds(s * cpt, cpt)],
                        cnt_out.at[pl.ds(c * n_cnt + s * cpt, cpt)])

    run = pl.kernel(
        body,
        out_type=[jax.ShapeDtypeStruct((_NC * n_cnt,), _F32)],
        mesh=mesh,
        scratch_types=dict(
            dst_v0=pltpu.VMEM((chunk,), jnp.int32),
            dst_v1=pltpu.VMEM((chunk,), jnp.int32),
            ones_v=pltpu.VMEM((1024,), _F32),
            zflat=pltpu.VMEM((1024,), _F32),
            cnt_sh=pltpu.VMEM_SHARED((n_cnt,), _F32),
        ),
        compiler_params=pltpu.CompilerParams(use_tc_tiling_on_sc=False))
    (cnt_out,) = run(dst)
    return cnt_out


def _hidden_body(agg_ref, cnt_ref, root_ref, bias_ref, h_ref):
    denom = jnp.maximum(cnt_ref[0] + cnt_ref[1], 1.0)
    h = (agg_ref[0] + agg_ref[1]) / denom + root_ref[...] + bias_ref[...]
    h_ref[...] = jnp.maximum(h, 0.0)


def _hidden(agg1p, cntp, root1, bias1, nb):
    n, h_dim = root1.shape
    return pl.pallas_call(
        _hidden_body,
        grid=(n // nb,),
        in_specs=[
            pl.BlockSpec((2, nb, h_dim), lambda i: (0, i, 0)),
            pl.BlockSpec((2, nb, 1), lambda i: (0, i, 0)),
            pl.BlockSpec((nb, h_dim), lambda i: (i, 0)),
            pl.BlockSpec((1, h_dim), lambda i: (0, 0)),
        ],
        out_specs=pl.BlockSpec((nb, h_dim), lambda i: (i, 0)),
        out_shape=jax.ShapeDtypeStruct((n, h_dim), _F32),
    )(agg1p, cntp, root1, bias1)


def _table2_body(h_ref, comp2_ref, basis2_ref, out_ref):
    r_dim, b_dim = comp2_ref.shape
    cols = []
    for r in range(r_dim):
        w2r = comp2_ref[r, 0] * basis2_ref[0]
        for b in range(1, b_dim):
            w2r = w2r + comp2_ref[r, b] * basis2_ref[b]
        cols.append(w2r)
    w_all = jnp.concatenate(cols, axis=1)
    out_ref[...] = jnp.dot(h_ref[...], w_all, preferred_element_type=_F32)


def _table2(h, comp2, basis2, nb):
    n, h_dim = h.shape
    r_dim = comp2.shape[0]
    c_dim = basis2.shape[2]
    b_dim = basis2.shape[0]
    out = pl.pallas_call(
        _table2_body,
        grid=(n // nb,),
        in_specs=[
            pl.BlockSpec((nb, h_dim), lambda i: (i, 0)),
            pl.BlockSpec(memory_space=pltpu.SMEM),
            pl.BlockSpec((b_dim, h_dim, c_dim), lambda i: (0, 0, 0)),
        ],
        out_specs=pl.BlockSpec((nb, r_dim * c_dim), lambda i: (i, 0)),
        out_shape=jax.ShapeDtypeStruct((n, r_dim * c_dim), _F32),
    )(h, comp2, basis2)
    return out.reshape(n * r_dim, c_dim)


def _out_body(agg_ref, cnt_ref, h_ref, root2_ref, bias_ref, out_ref):
    denom = jnp.maximum(cnt_ref[0] + cnt_ref[1], 1.0)
    o = (agg_ref[0] + agg_ref[1]) / denom
    o = o + jnp.dot(h_ref[...], root2_ref[...], preferred_element_type=_F32)
    o = o + bias_ref[...]
    m = jnp.max(o, axis=1, keepdims=True)
    e = o - m
    out_ref[...] = e - jnp.log(jnp.sum(jnp.exp(e), axis=1, keepdims=True))


def _final_out(agg2p, cntp, h, root2, bias2, nb):
    n, h_dim = h.shape
    c_dim = root2.shape[1]
    return pl.pallas_call(
        _out_body,
        grid=(n // nb,),
        in_specs=[
            pl.BlockSpec((2, nb, c_dim), lambda i: (0, i, 0)),
            pl.BlockSpec((2, nb, 1), lambda i: (0, i, 0)),
            pl.BlockSpec((nb, h_dim), lambda i: (i, 0)),
            pl.BlockSpec((h_dim, c_dim), lambda i: (0, 0)),
            pl.BlockSpec((1, c_dim), lambda i: (0, 0)),
        ],
        out_specs=pl.BlockSpec((nb, c_dim), lambda i: (i, 0)),
        out_shape=jax.ShapeDtypeStruct((n, c_dim), _F32),
    )(agg2p, cntp, h, root2, bias2)


def kernel(x, edge_index, edge_type, basis1, comp1, root1, bias1,
           basis2, comp2, root2, bias2):
    del x
    b_dim, n, h_dim = basis1.shape
    r_dim = comp1.shape[0]
    c_dim = basis2.shape[2]
    e = edge_type.shape[0]

    src = edge_index[0]
    dst = edge_index[1]
    gidx1 = edge_type * n + src
    gidx2 = src * r_dim + edge_type

    cntp = _sc_cnt(dst, n, chunk=1000)
    cnt3 = cntp.reshape(_NC, -1, 1)

    w1 = _materialize_weight1(basis1, comp1, nbb=500)

    (agg1p,) = _sc_agg(w1, gidx1, dst, n, h_dim, chunk=200, nbuf=3)

    h = _hidden(agg1p, cnt3, root1, bias1.reshape(1, h_dim), nb=2000)
    yw = _table2(h, comp2, basis2, nb=2000)

    (agg2p,) = _sc_agg(yw, gidx2, dst, n, c_dim, chunk=1000)

    return _final_out(agg2p, cnt3, h, root2, bias2.reshape(1, c_dim), nb=2000)
